# Initial kernel scaffold; baseline (speedup 1.0000x reference)
import jax
import jax.numpy as jnp
from jax.experimental import pallas as pl


def kernel(x, edge_index, batch, paper_count, W1, b1, W2, b2, W3, b3, lw1, lb1, lw2, lb2, lw3, lb3):
    raise NotImplementedError("write your pallas kernel here")


# speedup vs baseline: 12.1223x; 12.1223x over previous
import functools

import jax
import jax.numpy as jnp
from jax import lax
from jax.experimental import pallas as pl
from jax.experimental.pallas import tpu as pltpu
from jax.experimental.pallas import tpu_sc as plsc

N = 10000
E = 320000
D = 128
H = 128
G = 64
C = 10

NC = 2
NS = 16
NW = NC * NS
EPT = E // NW
BB = 128
NB = -(-EPT // BB)
EPAD = NB * BB
NPAD = 10240
RPT = NPAD // NS
TRASH = N

_f32 = jnp.float32


def _deg_body(dstp, degp, idx_v, ones_v, zb_v, acc_sp):
    c = lax.axis_index("c")
    s = lax.axis_index("s")
    wid = c * NS + s
    pltpu.sync_copy(dstp.at[wid], idx_v)
    for k in range(8):
        ones_v[pl.ds(k * 16, 16)] = jnp.ones((16,), _f32)

    def zf(i, _):
        zb_v[pl.ds(i * 16, 16)] = jnp.zeros((16,), _f32)
        return 0
    lax.fori_loop(0, RPT // 16, zf, 0)
    pltpu.sync_copy(zb_v, acc_sp.at[pl.ds(s * RPT, RPT)])
    plsc.subcore_barrier()

    def eb(j, _):
        pltpu.sync_copy(ones_v, acc_sp.at[idx_v.at[j]], add=True)
        return 0
    lax.fori_loop(0, NB, eb, 0)
    plsc.subcore_barrier()
    pltpu.sync_copy(acc_sp.at[pl.ds(s * RPT, RPT)], degp.at[c, pl.ds(s * RPT, RPT)])


@functools.lru_cache(maxsize=None)
def _sc_calls():
    mesh = plsc.VectorSubcoreMesh(core_axis_name="c", subcore_axis_name="s")
    deg = pl.kernel(
        _deg_body,
        out_type=jax.ShapeDtypeStruct((NC, NPAD), _f32),
        mesh=mesh,
        scratch_types=[
            pltpu.VMEM((NB, BB), jnp.int32),
            pltpu.VMEM((BB,), _f32),
            pltpu.VMEM((RPT,), _f32),
            pltpu.VMEM_SHARED((NPAD,), _f32),
        ],
    )
    prop = pl.kernel(
        _prop_body,
        out_type=jax.ShapeDtypeStruct((NC, NPAD, H), _f32),
        mesh=mesh,
        scratch_types=[
            pltpu.VMEM((NB, BB), jnp.int32),
            pltpu.VMEM((NB, BB), jnp.int32),
            pltpu.VMEM((BB, H), _f32),
            pltpu.VMEM((64, H), _f32),
            pltpu.VMEM_SHARED((NPAD, H), _f32),
        ],
    )
    return deg, prop


def _prop_body(y, srcp, dstp, outp, sidx, didx, rows, zb, acc_sp):
    c = lax.axis_index("c")
    s = lax.axis_index("s")
    wid = c * NS + s
    pltpu.sync_copy(srcp.at[wid], sidx)
    pltpu.sync_copy(dstp.at[wid], didx)

    def zf(i, _):
        for k in range(8):
            zb[i, pl.ds(k * 16, 16)] = jnp.zeros((16,), _f32)
        return 0
    lax.fori_loop(0, 64, zf, 0)

    def zcp(i, _):
        pltpu.sync_copy(zb, acc_sp.at[pl.ds(s * RPT + i * 64, 64)])
        return 0
    lax.fori_loop(0, RPT // 64, zcp, 0)
    plsc.subcore_barrier()

    def eb(j, _):
        pltpu.sync_copy(y.at[sidx.at[j]], rows)
        pltpu.sync_copy(rows, acc_sp.at[didx.at[j]], add=True)
        return 0
    lax.fori_loop(0, NB, eb, 0)
    plsc.subcore_barrier()
    pltpu.sync_copy(acc_sp.at[pl.ds(s * RPT, RPT)],
                    outp.at[c, pl.ds(s * RPT, RPT)])


BLK = 1000
NBLK = N // BLK


def _mm_body(x_ref, w_ref, o_ref):
    o_ref[...] = jnp.dot(x_ref[...], w_ref[...],
                         preferred_element_type=_f32)


_mm1 = pl.pallas_call(
    _mm_body,
    grid=(NBLK,),
    in_specs=[pl.BlockSpec((BLK, D), lambda i: (i, 0)),
              pl.BlockSpec((D, H), lambda i: (0, 0))],
    out_specs=pl.BlockSpec((BLK, H), lambda i: (i, 0)),
    out_shape=jax.ShapeDtypeStruct((N, H), _f32),
)


def _t0_body(degp_ref, xw_ref, dis_ref, dis2_ref, y_ref):
    deg = degp_ref[0] + degp_ref[1] + 1.0
    dis = lax.rsqrt(deg)
    dis2 = 1.0 / deg
    dis_ref[...] = dis
    dis2_ref[...] = dis2
    y_ref[...] = xw_ref[...] * dis


_t0 = pl.pallas_call(
    _t0_body,
    grid=(NBLK,),
    in_specs=[pl.BlockSpec((NC, BLK, 1), lambda i: (0, i, 0)),
              pl.BlockSpec((BLK, H), lambda i: (i, 0))],
    out_specs=[pl.BlockSpec((BLK, 1), lambda i: (i, 0)),
               pl.BlockSpec((BLK, 1), lambda i: (i, 0)),
               pl.BlockSpec((BLK, H), lambda i: (i, 0))],
    out_shape=[jax.ShapeDtypeStruct((N, 1), _f32),
               jax.ShapeDtypeStruct((N, 1), _f32),
               jax.ShapeDtypeStruct((N, H), _f32)],
)


def _blayer_body(with_next, with_cnt, *refs):
    if with_next:
        (acc_ref, xw_ref, dis_ref, dis2_ref, b_ref, bat_ref, w_ref,
         s_ref, *rest) = refs
        if with_cnt:
            cnt_ref, xwn_ref, yn_ref = rest
        else:
            xwn_ref, yn_ref = rest
    else:
        acc_ref, xw_ref, dis_ref, dis2_ref, b_ref, bat_ref, s_ref = refs
    i = pl.program_id(0)
    dis = dis_ref[...]
    a = acc_ref[0] + acc_ref[1]
    h = jnp.maximum(a * dis + xw_ref[...] * dis2_ref[...] + b_ref[...], 0.0)
    bat = bat_ref[0]
    gi = lax.broadcasted_iota(jnp.int32, (G, BLK), 0)
    oh = (gi == bat).astype(_f32)
    sc = jnp.dot(oh, h, preferred_element_type=_f32)

    @pl.when(i == 0)
    def _():
        s_ref[...] = jnp.zeros_like(s_ref)
        if with_next and with_cnt:
            cnt_ref[...] = jnp.zeros_like(cnt_ref)

    s_ref[...] += sc
    if with_next:
        if with_cnt:
            cnt_ref[...] += jnp.sum(oh, axis=1, keepdims=True)
        xwn = jnp.dot(h, w_ref[...], preferred_element_type=_f32)
        xwn_ref[...] = xwn
        yn_ref[...] = xwn * dis


def _make_blayer(with_next, with_cnt):
    in_specs = [
        pl.BlockSpec((NC, BLK, H), lambda i: (0, i, 0)),
        pl.BlockSpec((BLK, H), lambda i: (i, 0)),
        pl.BlockSpec((BLK, 1), lambda i: (i, 0)),
        pl.BlockSpec((BLK, 1), lambda i: (i, 0)),
        pl.BlockSpec((1, H), lambda i: (0, 0)),
        pl.BlockSpec((1, 1, BLK), lambda i: (i, 0, 0)),
    ]
    out_specs = [pl.BlockSpec((G, H), lambda i: (0, 0))]
    out_shape = [jax.ShapeDtypeStruct((G, H), _f32)]
    if with_next:
        in_specs.append(pl.BlockSpec((H, H), lambda i: (0, 0)))
        if with_cnt:
            out_specs.append(pl.BlockSpec((G, 1), lambda i: (0, 0)))
            out_shape.append(jax.ShapeDtypeStruct((G, 1), _f32))
        out_specs += [pl.BlockSpec((BLK, H), lambda i: (i, 0)),
                      pl.BlockSpec((BLK, H), lambda i: (i, 0))]
        out_shape += [jax.ShapeDtypeStruct((N, H), _f32),
                      jax.ShapeDtypeStruct((N, H), _f32)]
    return pl.pallas_call(
        functools.partial(_blayer_body, with_next, with_cnt),
        grid=(NBLK,),
        in_specs=in_specs,
        out_specs=out_specs,
        out_shape=out_shape,
    )


_b_first = _make_blayer(True, True)
_b_mid = _make_blayer(True, False)
_b_last = _make_blayer(False, False)


def _head_body(s1_ref, s2_ref, s3_ref, cnt_ref, pc_ref,
               lw1_ref, lb1_ref, lw2_ref, lb2_ref, lw3_ref, lb3_ref, o_ref):
    inv = 1.0 / jnp.maximum(cnt_ref[...], 1.0)
    g = (jnp.maximum(s1_ref[...] * inv, 0.0)
         + jnp.maximum(s2_ref[...] * inv, 0.0)
         + jnp.maximum(s3_ref[...] * inv, 0.0))
    g1 = jnp.maximum(
        jnp.dot(g, lw1_ref[...], preferred_element_type=_f32) + lb1_ref[...],
        0.0)
    l2 = lw2_ref[...]
    g2 = jnp.maximum(
        jnp.dot(g1, l2[:H // 2], preferred_element_type=_f32)
        + pc_ref[...] * l2[H // 2:H // 2 + 1]
        + lb2_ref[...],
        0.0)
    z = jnp.dot(g2, lw3_ref[...], preferred_element_type=_f32) + lb3_ref[...]
    m = jnp.max(z, axis=-1, keepdims=True)
    e = jnp.exp(z - m)
    o_ref[...] = z - m - jnp.log(jnp.sum(e, axis=-1, keepdims=True))


_head = pl.pallas_call(
    _head_body,
    grid=(1,),
    in_specs=[pl.BlockSpec((G, H), lambda i: (0, 0)),
              pl.BlockSpec((G, H), lambda i: (0, 0)),
              pl.BlockSpec((G, H), lambda i: (0, 0)),
              pl.BlockSpec((G, 1), lambda i: (0, 0)),
              pl.BlockSpec((G, 1), lambda i: (0, 0)),
              pl.BlockSpec((H, H // 2), lambda i: (0, 0)),
              pl.BlockSpec((1, H // 2), lambda i: (0, 0)),
              pl.BlockSpec((H // 2 + 1, H // 4), lambda i: (0, 0)),
              pl.BlockSpec((1, H // 4), lambda i: (0, 0)),
              pl.BlockSpec((H // 4, C), lambda i: (0, 0)),
              pl.BlockSpec((1, C), lambda i: (0, 0))],
    out_specs=pl.BlockSpec((G, C), lambda i: (0, 0)),
    out_shape=jax.ShapeDtypeStruct((G, C), _f32),
)


def kernel(x, edge_index, batch, paper_count, W1, b1, W2, b2, W3, b3,
           lw1, lb1, lw2, lb2, lw3, lb3):
    pad = EPAD - EPT
    src = edge_index[0].reshape(NW, EPT)
    dst = edge_index[1].reshape(NW, EPT)
    srcp = jnp.concatenate(
        [src, jnp.zeros((NW, pad), jnp.int32)], axis=1).reshape(NW, NB, BB)
    dstp = jnp.concatenate(
        [dst, jnp.full((NW, pad), TRASH, jnp.int32)], axis=1).reshape(NW, NB, BB)
    bat3 = batch.reshape(NBLK, 1, BLK)
    _deg_call, _prop_call = _sc_calls()

    degp = _deg_call(dstp).reshape(NC, NPAD, 1)
    xw1 = _mm1(x, W1)
    dis, dis2, y1 = _t0(degp, xw1)

    acc1 = _prop_call(y1, srcp, dstp)
    s1, cnt, xw2, y2 = _b_first(acc1, xw1, dis, dis2, b1.reshape(1, H),
                                bat3, W2)
    acc2 = _prop_call(y2, srcp, dstp)
    s2, xw3, y3 = _b_mid(acc2, xw2, dis, dis2, b2.reshape(1, H), bat3, W3)
    acc3 = _prop_call(y3, srcp, dstp)
    s3 = _b_last(acc3, xw3, dis, dis2, b3.reshape(1, H), bat3)
    if isinstance(s3, (list, tuple)):
        s3 = s3[0]

    return _head(s1, s2, s3, cnt, paper_count.reshape(G, 1),
                 lw1, lb1.reshape(1, H // 2), lw2, lb2.reshape(1, H // 4),
                 lw3, lb3.reshape(1, C))
